# baseline (speedup 1.0000x reference)
import functools

import jax
import jax.numpy as jnp
from jax import lax
from jax.experimental import pallas as pl
from jax.experimental.pallas import tpu as pltpu
from jax.experimental.pallas import tpu_sc as plsc

N_NODES = 10000
N_EDGES = 320000
D_IN = 128
D_H = 192
N_LAYERS = 4
NUM_GRAPHS = 64
EPS = 1e-5

NC = 2
NS = 16
CHUNK = 128
N_PAD = 10240
CPW = 157
E_PAD = NS * CPW * CHUNK


@functools.lru_cache(maxsize=None)
def _make_edge_agg(dh):
    rows_t = N_PAD // NS
    mesh = plsc.VectorSubcoreMesh(core_axis_name="c", subcore_axis_name="s")

    @functools.partial(
        pl.kernel,
        out_type=jax.ShapeDtypeStruct((NC, N_PAD, dh), jnp.float32),
        mesh=mesh,
        scratch_types=[
            pltpu.VMEM((CPW, CHUNK), jnp.int32),
            pltpu.VMEM((CHUNK,), jnp.int32),
            pltpu.VMEM((CHUNK,), jnp.int32),
            pltpu.VMEM((CHUNK,), jnp.int32),
            pltpu.VMEM((CHUNK, dh), jnp.float32),
            pltpu.VMEM((CHUNK, dh), jnp.float32),
            pltpu.VMEM((CHUNK, dh), jnp.float32),
            pltpu.VMEM_SHARED((N_PAD, dh), jnp.float32),
            pltpu.SemaphoreType.DMA,
            pltpu.SemaphoreType.DMA,
            pltpu.SemaphoreType.DMA,
            pltpu.SemaphoreType.DMA,
            pltpu.SemaphoreType.DMA,
            pltpu.SemaphoreType.DMA,
            pltpu.SemaphoreType.DMA,
            pltpu.SemaphoreType.DMA,
            pltpu.SemaphoreType.DMA,
        ],
        compiler_params=pltpu.CompilerParams(use_tc_tiling_on_sc=False),
    )
    def edge_kernel(src_hbm, dst_hbm, hlo_hbm, hhi_hbm, zeros_hbm, out_hbm,
                    src_v, db0, db1, db2, buf0, buf1, buf2, acc_sh,
                    gs0, gs1, gs2, ss0, ss1, ss2, ds0, ds1, ds2):
        bufs = (buf0, buf1, buf2)
        dbufs = (db0, db1, db2)
        gsems = (gs0, gs1, gs2)
        ssems = (ss0, ss1, ss2)
        dsems = (ds0, ds1, ds2)
        c = lax.axis_index("c")
        s = lax.axis_index("s")
        pltpu.sync_copy(zeros_hbm, acc_sh.at[pl.ds(s * rows_t, rows_t)])
        pltpu.sync_copy(src_hbm.at[pl.ds(s * CPW, CPW)], src_v)
        plsc.subcore_barrier()

        def gather(g, b):
            pltpu.async_copy(dst_hbm.at[s * CPW + g], dbufs[b], dsems[b])

            @pl.when(c == 0)
            def _():
                pltpu.async_copy(hlo_hbm.at[src_v.at[g]], bufs[b], gsems[b])

            @pl.when(c == 1)
            def _():
                pltpu.async_copy(hhi_hbm.at[src_v.at[g]], bufs[b], gsems[b])

        def wait_gather(b):
            pltpu.make_async_copy(
                dst_hbm.at[0], dbufs[b], dsems[b]).wait()
            pltpu.make_async_copy(
                hlo_hbm.at[src_v.at[0]], bufs[b], gsems[b]).wait()

        def scatter(g, b):
            pltpu.async_copy(bufs[b], acc_sh.at[dbufs[b]], ssems[b],
                             add=True)

        def wait_scatter(b):
            pltpu.make_async_copy(
                bufs[b], acc_sh.at[dbufs[b]], ssems[b]).wait()

        gather(0, 0)
        gather(1, 1)
        nt = (CPW - 1) // 3

        @pl.loop(0, nt)
        def _(t):
            for b in range(3):
                wait_gather(b)
                g = 3 * t + b
                scatter(g, b)
                prev = (b - 1) % 3
                if b == 0:
                    @pl.when(t > 0)
                    def _():
                        wait_scatter(prev)
                        gather(g + 2, prev)

                    @pl.when(t == 0)
                    def _():
                        gather(g + 2, prev)
                elif b == 1:
                    wait_scatter(prev)
                    gather(g + 2, prev)
                else:
                    wait_scatter(prev)

                    @pl.when(t < nt - 1)
                    def _():
                        gather(g + 2, prev)

        wait_scatter(2)
        wait_gather(0)
        pltpu.sync_copy(bufs[0], acc_sh.at[dbufs[0]], add=True)

        plsc.subcore_barrier()
        pltpu.sync_copy(acc_sh.at[pl.ds(s * rows_t, rows_t)],
                        out_hbm.at[c, pl.ds(s * rows_t, rows_t)])

    return edge_kernel


@functools.lru_cache(maxsize=None)
def _make_layer_dense(dh):
    br = 1000
    nb = N_NODES // br
    inv_n = 1.0 / N_NODES
    dho = D_H // 2

    def body(acc_ref, hlo_ref, hhi_ref, wrel_ref, brel_ref, wroot_ref,
             gb_ref, lo_ref, hi_ref, h2_scr, stats_scr):
        p = pl.program_id(0)
        b = pl.program_id(1)

        @pl.when(p == 0)
        def _():
            h2 = (jnp.dot(acc_ref[0], wrel_ref[0],
                          preferred_element_type=jnp.float32)
                  + jnp.dot(acc_ref[1], wrel_ref[1],
                            preferred_element_type=jnp.float32)
                  + jnp.dot(hlo_ref[...], wroot_ref[0],
                            preferred_element_type=jnp.float32)
                  + jnp.dot(hhi_ref[...], wroot_ref[1],
                            preferred_element_type=jnp.float32)
                  + brel_ref[...])
            h2_scr[pl.ds(b * br, br), :] = h2

            @pl.when(b == 0)
            def _():
                stats_scr[...] = jnp.zeros_like(stats_scr)

            stats_scr[0:1, :] += jnp.sum(h2, axis=0, keepdims=True)
            stats_scr[1:2, :] += jnp.sum(h2 * h2, axis=0, keepdims=True)

        @pl.when(p == 1)
        def _():
            mean = stats_scr[0:1, :] * inv_n
            var = stats_scr[1:2, :] * inv_n - mean * mean
            scale = gb_ref[0:1, :] / jnp.sqrt(var + EPS)
            shift = gb_ref[1:2, :] - mean * scale
            res = jnp.maximum(
                h2_scr[pl.ds(b * br, br), :] * scale + shift, 0.0)
            lo_ref[...] = res[:, :dho]
            hi_ref[...] = res[:, dho:]

    return pl.pallas_call(
        body,
        grid=(2, nb),
        in_specs=[
            pl.BlockSpec((NC, br, dh), lambda p, b: (0, b * (1 - p), 0)),
            pl.BlockSpec((br, dh), lambda p, b: (b * (1 - p), 0)),
            pl.BlockSpec((br, dh), lambda p, b: (b * (1 - p), 0)),
            pl.BlockSpec((NC, dh, D_H), lambda p, b: (0, 0, 0)),
            pl.BlockSpec((1, D_H), lambda p, b: (0, 0)),
            pl.BlockSpec((NC, dh, D_H), lambda p, b: (0, 0, 0)),
            pl.BlockSpec((2, D_H), lambda p, b: (0, 0)),
        ],
        out_specs=[
            pl.BlockSpec((br, dho), lambda p, b: (b * p, 0)),
            pl.BlockSpec((br, dho), lambda p, b: (b * p, 0)),
        ],
        out_shape=[
            jax.ShapeDtypeStruct((N_NODES, dho), jnp.float32),
            jax.ShapeDtypeStruct((N_NODES, dho), jnp.float32),
        ],
        scratch_shapes=[
            pltpu.VMEM((N_NODES, D_H), jnp.float32),
            pltpu.VMEM((2, D_H), jnp.float32),
        ],
    )


def _make_pool_head():
    dh = D_H // 2

    def body(hlo_ref, hhi_ref, bc_ref, w1_ref, b1_ref, w2_ref, b2_ref,
             wo_ref, bo_ref, out_ref):
        giota = lax.broadcasted_iota(
            jnp.int32, (N_NODES, NUM_GRAPHS), 1).astype(jnp.float32)
        eqt = (bc_ref[...] == giota).astype(jnp.float32)
        ones = jnp.zeros((N_NODES, 1), jnp.float32) + 1.0
        dn = (((0,), (0,)), ((), ()))
        counts = lax.dot_general(eqt, ones, dn,
                                 preferred_element_type=jnp.float32)
        inv = 1.0 / jnp.maximum(counts, 1.0)
        g0 = lax.dot_general(eqt, hlo_ref[...], dn,
                             preferred_element_type=jnp.float32) * inv
        g1 = lax.dot_general(eqt, hhi_ref[...], dn,
                             preferred_element_type=jnp.float32) * inv
        gp = jnp.concatenate(
            [g0, jnp.zeros((NUM_GRAPHS, 128 - dh), jnp.float32),
             g1, jnp.zeros((NUM_GRAPHS, 128 - dh), jnp.float32)], axis=1)
        hh = jnp.maximum(
            jnp.dot(gp, w1_ref[...], preferred_element_type=jnp.float32)
            + b1_ref[...], 0.0)
        hh = jnp.concatenate(
            [hh, jnp.zeros((NUM_GRAPHS, 256 - D_H), jnp.float32)], axis=1)
        hh = jnp.dot(hh, w2_ref[...],
                     preferred_element_type=jnp.float32) + b2_ref[...]
        hh = jnp.concatenate(
            [hh, jnp.zeros((NUM_GRAPHS, 256 - D_H), jnp.float32)], axis=1)
        out_ref[...] = jnp.dot(hh, wo_ref[...],
                               preferred_element_type=jnp.float32) + bo_ref[...]

    return pl.pallas_call(
        body,
        out_shape=jax.ShapeDtypeStruct((NUM_GRAPHS, 1), jnp.float32),
    )


_pool_head = _make_pool_head()


def _pad_rows(w, rows):
    return jnp.pad(w, ((0, rows - w.shape[0]), (0, 0)))


def _split2(w):
    d = w.shape[0]
    return jnp.stack([w[:d // 2], w[d // 2:]])


def kernel(x, edge_index, batch, params):
    src = edge_index[0].astype(jnp.int32)
    dst = edge_index[1].astype(jnp.int32)
    batch_c = batch.astype(jnp.float32).reshape(N_NODES, 1)

    npad_e = E_PAD - N_EDGES
    src2d = jnp.concatenate(
        [src, jnp.zeros((npad_e,), jnp.int32)]).reshape(NS * CPW, CHUNK)
    dst2d = jnp.concatenate(
        [dst, jnp.full((npad_e,), N_NODES, jnp.int32)]).reshape(NS * CPW, CHUNK)

    hlo, hhi = x[:, :D_IN // 2], x[:, D_IN // 2:]
    d = D_IN
    for i in range(N_LAYERS):
        p = params[f'conv{i}']
        dh = d // 2
        zeros = jnp.zeros((N_PAD // NS, dh), dtype=jnp.float32)
        acc2 = _make_edge_agg(dh)(src2d, dst2d, hlo, hhi, zeros)
        gb = jnp.stack([p['gamma'], p['beta']])
        hlo, hhi = _make_layer_dense(dh)(
            acc2, hlo, hhi, _split2(p['W_rel']), p['b_rel'].reshape(1, D_H),
            _split2(p['W_root']), gb)
        d = D_H

    hd = params['head']
    dh = D_H // 2
    w1p = jnp.concatenate(
        [hd['W1'][:dh], jnp.zeros((128 - dh, D_H), jnp.float32),
         hd['W1'][dh:], jnp.zeros((128 - dh, D_H), jnp.float32)], axis=0)
    out = _pool_head(
        hlo, hhi, batch_c,
        w1p, hd['b1'].reshape(1, D_H),
        _pad_rows(hd['W2'], 256), hd['b2'].reshape(1, D_H),
        _pad_rows(params['out']['W'], 256), params['out']['b'].reshape(1, 1))
    return out.reshape(-1)
